# baseline (speedup 1.0000x reference)
import functools

import jax
import jax.numpy as jnp
from jax import lax
from jax.experimental import pallas as pl
from jax.experimental.pallas import tpu as pltpu
from jax.experimental.pallas import tpu_sc as plsc

_D = 1000
_B, _T = 1024, 50
_NC, _NS = 2, 16
_NW = _NC * _NS
_BPW = _B // _NW
_PPB = 7 * 8 * 8


def _sc_gather(idx, table128):
  mesh = plsc.VectorSubcoreMesh(core_axis_name="c", subcore_axis_name="s")

  @functools.partial(
      pl.kernel,
      out_type=jax.ShapeDtypeStruct((_B, _T, _D), jnp.float32),
      mesh=mesh,
      scratch_types=[
          pltpu.VMEM((_BPW * _PPB,), jnp.int32),
          pltpu.VMEM((_PPB, 128), jnp.float32),
          pltpu.VMEM((_PPB, 128), jnp.float32),
          pltpu.SemaphoreType.DMA,
          pltpu.SemaphoreType.DMA,
          pltpu.SemaphoreType.DMA,
          pltpu.SemaphoreType.DMA,
      ],
  )
  def body(idx_hbm, table_hbm, out_hbm, idx_v, rows0, rows1, g0, g1, s0, s1):
    wid = lax.axis_index("s") * _NC + lax.axis_index("c")
    b0 = wid * _BPW
    pltpu.sync_copy(idx_hbm.at[pl.ds(b0 * _PPB, _BPW * _PPB)], idx_v)

    bufs = (rows0, rows1)
    gsems = (g0, g1)
    ssems = (s0, s1)

    def gather(c, p):
      off = pl.multiple_of(c * _PPB, _PPB)
      return pltpu.make_async_copy(
          table_hbm.at[idx_v.at[pl.ds(off, _PPB)]], bufs[p], gsems[p])

    def _tile_copies(c, p):
      cps = []
      tail = _D - 128 * 7
      for g in range(7):
        rows = 8 if g < 6 else 2
        for tc in range(7):
          cps.append(pltpu.make_async_copy(
              bufs[p].at[pl.ds(8 * (8 * g + tc), rows), :],
              out_hbm.at[b0 + c, pl.ds(8 * g, rows), pl.ds(128 * tc, 128)],
              ssems[p]))
        for k in range(rows):
          cps.append(pltpu.make_async_copy(
              bufs[p].at[8 * (8 * g + 7) + k, pl.ds(0, tail)],
              out_hbm.at[b0 + c, 8 * g + k, pl.ds(128 * 7, tail)],
              ssems[p]))
      return cps

    class _Scatter:
      def __init__(self, c, p):
        self.cps = _tile_copies(c, p)

      def start(self):
        for cp in self.cps:
          cp.start()

      def wait(self):
        for cp in self.cps:
          cp.wait()

    def scatter(c, p):
      return _Scatter(c, p)

    gather(0, 0).start()
    gather(1, 1).start()

    def step(jj, carry):
      c0 = 2 * jj
      gather(c0, 0).wait()
      scatter(c0, 0).start()
      gather(c0 + 1, 1).wait()
      scatter(c0 + 1, 1).start()
      scatter(c0, 0).wait()
      gather(c0 + 2, 0).start()
      scatter(c0 + 1, 1).wait()
      gather(c0 + 3, 1).start()
      return carry

    lax.fori_loop(0, _BPW // 2 - 1, step, 0)

    cl = _BPW - 2
    gather(cl, 0).wait()
    scatter(cl, 0).start()
    gather(cl + 1, 1).wait()
    scatter(cl + 1, 1).start()
    scatter(cl, 0).wait()
    scatter(cl + 1, 1).wait()

  return body(idx, table128)


def kernel(x, prob):
  table128 = jnp.pad(prob, ((0, 0), (0, 24))).reshape(8000, 128)
  xp = jnp.pad(x, ((0, 0), (0, 6)))
  r3 = xp.reshape(_B, 7, 1, 8)
  tc = jnp.arange(8, dtype=x.dtype).reshape(1, 1, 8, 1)
  idx = (8 * r3 + tc).reshape(-1).astype(jnp.int32)
  return _sc_gather(idx, table128)


# speedup vs baseline: 1.1079x; 1.1079x over previous
import functools

import jax
import jax.numpy as jnp
from jax import lax
from jax.experimental import pallas as pl
from jax.experimental.pallas import tpu as pltpu
from jax.experimental.pallas import tpu_sc as plsc

_D = 1000
_B, _T = 1024, 50
_NC, _NS = 2, 16
_NW = _NC * _NS
_CHUNKS = 4


def _sc_gather(x, prob):
  nb = x.shape[0]
  bpw = nb // _NW
  mesh = plsc.VectorSubcoreMesh(core_axis_name="c", subcore_axis_name="s")

  @functools.partial(
      pl.kernel,
      out_type=jax.ShapeDtypeStruct((nb, _T, _D), jnp.float32),
      mesh=mesh,
      scratch_types=[
          pltpu.VMEM((bpw, _T), jnp.int32),
          pltpu.VMEM((_T, _D), jnp.float32),
          pltpu.VMEM((_T, _D), jnp.float32),
          pltpu.SemaphoreType.DMA,
          pltpu.SemaphoreType.DMA,
          pltpu.SemaphoreType.DMA,
          pltpu.SemaphoreType.DMA,
      ],
      compiler_params=pltpu.CompilerParams(use_tc_tiling_on_sc=False),
  )
  def body(idx_hbm, table_hbm, out_hbm, idx_v, rows0, rows1, g0, g1, s0, s1):
    wid = lax.axis_index("s") * _NC + lax.axis_index("c")
    b0 = wid * bpw
    pltpu.sync_copy(idx_hbm.at[pl.ds(b0, bpw)], idx_v)

    bufs = (rows0, rows1)
    gsems = (g0, g1)
    ssems = (s0, s1)

    def gather(c, p):
      return pltpu.make_async_copy(
          table_hbm.at[idx_v.at[c]], bufs[p], gsems[p])

    def scatter(c, p):
      return pltpu.make_async_copy(bufs[p], out_hbm.at[b0 + c], ssems[p])

    gather(0, 0).start()
    gather(1, 1).start()

    def step(jj, carry):
      c0 = 2 * jj
      gather(c0, 0).wait()
      scatter(c0, 0).start()
      gather(c0 + 1, 1).wait()
      scatter(c0 + 1, 1).start()
      scatter(c0, 0).wait()
      gather(c0 + 2, 0).start()
      scatter(c0 + 1, 1).wait()
      gather(c0 + 3, 1).start()
      return carry

    lax.fori_loop(0, bpw // 2 - 1, step, 0)

    cl = bpw - 2
    gather(cl, 0).wait()
    scatter(cl, 0).start()
    gather(cl + 1, 1).wait()
    scatter(cl + 1, 1).start()
    scatter(cl, 0).wait()
    scatter(cl + 1, 1).wait()

  return body(x, prob)


def kernel(x, prob):
  step = _B // _CHUNKS
  parts = [_sc_gather(x[i * step:(i + 1) * step], prob)
           for i in range(_CHUNKS)]
  return jnp.concatenate(parts, axis=0)
